# baseline (speedup 1.0000x reference)
import jax
import jax.numpy as jnp
from jax.experimental import pallas as pl
from jax.experimental.pallas import tpu as pltpu

_TEMP = 0.1
_N = 2048
_BLK = 256
_STEPS = _N // _BLK


def _valuations(m):
    v = jnp.zeros(m.shape, dtype=jnp.float32)
    for _ in range(16):
        div = (m > 0) & (m % 3 == 0)
        v = v + div.astype(jnp.float32)
        m = jnp.where(div, m // 3, m)
    return v


def _pair_kernel(z_ref, bi_ref, zi_ref, bii_ref, loss_ref, acc_ref):
    step = pl.program_id(0)

    z = z_ref[...]
    radii = jnp.sqrt(jnp.sum(z * z, axis=1))
    v = _valuations(bi_ref[...])

    zi = zi_ref[...]
    ri = jnp.sqrt(jnp.sum(zi * zi, axis=1))
    vi = _valuations(bii_ref[...])

    v_diff = vi[:, None] - v[None, :]
    r_diff = radii[None, :] - ri[:, None]
    mask = (v_diff != 0.0).astype(jnp.float32)
    s = jnp.sign(v_diff)
    viol = jax.nn.sigmoid(-(s * r_diff) / _TEMP)
    w = jnp.abs(v_diff)
    wsum = jnp.sum(viol * w * mask)
    csum = jnp.sum(mask)

    @pl.when(step == 0)
    def _():
        acc_ref[0, 0] = 0.0
        acc_ref[0, 1] = 0.0

    acc_ref[0, 0] += wsum
    acc_ref[0, 1] += csum

    @pl.when(step == _STEPS - 1)
    def _():
        loss_ref[0, 0] = acc_ref[0, 0] / jnp.maximum(acc_ref[0, 1], 1.0)


def kernel(z_hyp, batch_indices):
    loss = pl.pallas_call(
        _pair_kernel,
        grid=(_STEPS,),
        in_specs=[
            pl.BlockSpec((_N, 128), lambda i: (0, 0)),
            pl.BlockSpec((_N,), lambda i: (0,)),
            pl.BlockSpec((_BLK, 128), lambda i: (i, 0)),
            pl.BlockSpec((_BLK,), lambda i: (i,)),
        ],
        out_specs=pl.BlockSpec(
            block_shape=(1, 1),
            index_map=lambda i: (0, 0),
            memory_space=pltpu.SMEM,
        ),
        out_shape=jax.ShapeDtypeStruct((1, 1), jnp.float32),
        scratch_shapes=[pltpu.SMEM((1, 2), jnp.float32)],
    )(z_hyp, batch_indices, z_hyp, batch_indices)
    return loss[0, 0]


# speedup vs baseline: 56.0130x; 56.0130x over previous
import jax
import jax.numpy as jnp
from jax.experimental import pallas as pl
from jax.experimental.pallas import tpu as pltpu

_TEMP = 0.1
_N = 2048
_BLKJ = 256
_STEPS = _N // _BLKJ
_NBINS = 16


def _stage1(z_ref, bi_ref, rcol_ref, vrow_ref, const_ref):
    z = z_ref[...]
    rcol_ref[...] = jnp.sqrt(jnp.sum(z * z, axis=1, keepdims=True)) * (1.0 / _TEMP)

    m = bi_ref[...].astype(jnp.float32)
    v = jnp.zeros(m.shape, dtype=jnp.float32)
    for _ in range(13):
        q = jnp.round(m * (1.0 / 3.0))
        div = (m > 0.0) & (q * 3.0 == m)
        v = v + div.astype(jnp.float32)
        m = jnp.where(div, q, m)
    vrow_ref[...] = v

    bins = jax.lax.broadcasted_iota(jnp.int32, (_NBINS, 1), 0).astype(jnp.float32)
    n_b = jnp.sum((bins == v).astype(jnp.float32), axis=1, keepdims=True)
    w_b = jnp.sum(jnp.maximum(bins - v, 0.0), axis=1, keepdims=True)
    const_ref[0, 0] = jnp.sum(n_b * w_b)
    const_ref[0, 1] = float(_N * _N) - jnp.sum(n_b * n_b)
    const_ref[0, 2] = jnp.sum(v)


def _stage2(const_ref, rj_ref, rrow_ref, vrow_ref, out_ref, sacc_ref):
    step = pl.program_id(0)

    x = rrow_ref[...] - rj_ref[...]
    colsum = jnp.sum(jax.nn.sigmoid(x), axis=0, keepdims=True)

    @pl.when(step == 0)
    def _():
        sacc_ref[...] = jnp.zeros_like(sacc_ref)

    sacc_ref[...] += colsum

    @pl.when(step == _STEPS - 1)
    def _():
        v = vrow_ref[...]
        num = (2.0 * jnp.sum(v * sacc_ref[...])
               - float(_N) * const_ref[0, 2] + const_ref[0, 0])
        out_ref[0, 0] = num / jnp.maximum(const_ref[0, 1], 1.0)


def kernel(z_hyp, batch_indices):
    rcol, vrow, consts = pl.pallas_call(
        _stage1,
        in_specs=[
            pl.BlockSpec((_N, 128), lambda: (0, 0)),
            pl.BlockSpec((1, _N), lambda: (0, 0)),
        ],
        out_specs=[
            pl.BlockSpec((_N, 1), lambda: (0, 0)),
            pl.BlockSpec((1, _N), lambda: (0, 0)),
            pl.BlockSpec(block_shape=(1, 4), index_map=lambda: (0, 0),
                         memory_space=pltpu.SMEM),
        ],
        out_shape=[
            jax.ShapeDtypeStruct((_N, 1), jnp.float32),
            jax.ShapeDtypeStruct((1, _N), jnp.float32),
            jax.ShapeDtypeStruct((1, 4), jnp.float32),
        ],
    )(z_hyp, batch_indices.reshape(1, _N))

    rrow = rcol.reshape(1, _N)

    loss = pl.pallas_call(
        _stage2,
        grid=(_STEPS,),
        in_specs=[
            pl.BlockSpec(block_shape=(1, 4), index_map=lambda i: (0, 0),
                         memory_space=pltpu.SMEM),
            pl.BlockSpec((_BLKJ, 1), lambda i: (i, 0)),
            pl.BlockSpec((1, _N), lambda i: (0, 0)),
            pl.BlockSpec((1, _N), lambda i: (0, 0)),
        ],
        out_specs=pl.BlockSpec(block_shape=(1, 1), index_map=lambda i: (0, 0),
                               memory_space=pltpu.SMEM),
        out_shape=jax.ShapeDtypeStruct((1, 1), jnp.float32),
        scratch_shapes=[pltpu.VMEM((1, _N), jnp.float32)],
    )(consts, rcol, rrow, vrow)

    return loss[0, 0]


# speedup vs baseline: 138.5626x; 2.4738x over previous
import jax
import jax.numpy as jnp
from jax.experimental import pallas as pl
from jax.experimental.pallas import tpu as pltpu

_TEMP = 0.1
_N = 2048
_NBINS = 16


def _rank_loss_kernel(z_ref, bi_ref, out_ref):
    z = z_ref[...]
    rcol = jnp.sqrt(jnp.sum(z * z, axis=1, keepdims=True)) * (0.5 / _TEMP)
    rrow = jnp.transpose(rcol, (1, 0))

    m = bi_ref[...].astype(jnp.float32)
    v = jnp.zeros(m.shape, dtype=jnp.float32)
    for _ in range(13):
        q = jnp.round(m * (1.0 / 3.0))
        div = (m > 0.0) & (q * 3.0 == m)
        v = v + div.astype(jnp.float32)
        m = jnp.where(div, q, m)

    bins = jax.lax.broadcasted_iota(jnp.int32, (_NBINS, 1), 0).astype(jnp.float32)
    n_b = jnp.sum((bins == v).astype(jnp.float32), axis=1, keepdims=True)
    w_b = jnp.sum(jnp.maximum(bins - v, 0.0), axis=1, keepdims=True)
    c_const = jnp.sum(n_b * w_b)
    denom = float(_N * _N) - jnp.sum(n_b * n_b)

    x = rrow - rcol
    tsum = jnp.sum(jnp.tanh(x), axis=0, keepdims=True)
    num = jnp.sum(v * tsum) + c_const
    out_ref[0, 0] = num / jnp.maximum(denom, 1.0)


def kernel(z_hyp, batch_indices):
    loss = pl.pallas_call(
        _rank_loss_kernel,
        in_specs=[
            pl.BlockSpec((_N, 128), lambda: (0, 0)),
            pl.BlockSpec((1, _N), lambda: (0, 0)),
        ],
        out_specs=pl.BlockSpec(block_shape=(1, 1), index_map=lambda: (0, 0),
                               memory_space=pltpu.SMEM),
        out_shape=jax.ShapeDtypeStruct((1, 1), jnp.float32),
    )(z_hyp, batch_indices.reshape(1, _N))
    return loss[0, 0]


# speedup vs baseline: 155.1863x; 1.1200x over previous
import jax
import jax.numpy as jnp
from jax.experimental import pallas as pl
from jax.experimental.pallas import tpu as pltpu

_TEMP = 0.1
_N = 2048
_NBINS = 16


def _rank_loss_kernel(z_ref, bi_ref, out_ref):
    z = z_ref[...]
    rcol = jnp.sqrt(jnp.sum(z * z, axis=1, keepdims=True)) * (0.5 / _TEMP)
    rrow = jnp.transpose(rcol, (1, 0))

    m = bi_ref[...].astype(jnp.float32)
    v = jnp.zeros(m.shape, dtype=jnp.float32)
    for _ in range(13):
        q = jnp.round(m * (1.0 / 3.0))
        div = (m > 0.0) & (q * 3.0 == m)
        v = v + div.astype(jnp.float32)
        m = jnp.where(div, q, m)

    bins = jax.lax.broadcasted_iota(jnp.int32, (_NBINS, 1), 0).astype(jnp.float32)
    n_b = jnp.sum((bins == v).astype(jnp.float32), axis=1, keepdims=True)
    w_b = jnp.sum(jnp.maximum(bins - v, 0.0), axis=1, keepdims=True)
    c_const = jnp.sum(n_b * w_b)
    denom = float(_N * _N) - jnp.sum(n_b * n_b)

    x = rrow - rcol
    ones = jnp.ones((1, _N), dtype=jnp.float32)
    tsum = jax.lax.dot_general(
        ones, jnp.tanh(x), (((1,), (0,)), ((), ())),
        preferred_element_type=jnp.float32)
    num = jnp.sum(v * tsum) + c_const
    out_ref[0, 0] = num / jnp.maximum(denom, 1.0)


def kernel(z_hyp, batch_indices):
    loss = pl.pallas_call(
        _rank_loss_kernel,
        in_specs=[
            pl.BlockSpec((_N, 128), lambda: (0, 0)),
            pl.BlockSpec((1, _N), lambda: (0, 0)),
        ],
        out_specs=pl.BlockSpec(block_shape=(1, 1), index_map=lambda: (0, 0),
                               memory_space=pltpu.SMEM),
        out_shape=jax.ShapeDtypeStruct((1, 1), jnp.float32),
    )(z_hyp, batch_indices.reshape(1, _N))
    return loss[0, 0]


# speedup vs baseline: 171.2763x; 1.1037x over previous
import jax
import jax.numpy as jnp
from jax.experimental import pallas as pl
from jax.experimental.pallas import tpu as pltpu

_TEMP = 0.1
_N = 2048
_NBINS = 16
_K = 4
_H = _N // _K


def _rank_loss_kernel(z_ref, bi_ref, out_ref):
    z = z_ref[...]
    rcol = jnp.sqrt(jnp.sum(z * z, axis=1, keepdims=True)) * (0.5 / _TEMP)
    rrow = jnp.transpose(rcol, (1, 0))

    m = bi_ref[...].astype(jnp.float32)
    v = jnp.zeros(m.shape, dtype=jnp.float32)
    for _ in range(13):
        q = jnp.round(m * (1.0 / 3.0))
        div = (m > 0.0) & (q * 3.0 == m)
        v = v + div.astype(jnp.float32)
        m = jnp.where(div, q, m)

    bins = jax.lax.broadcasted_iota(jnp.int32, (_NBINS, 1), 0).astype(jnp.float32)
    n_b = jnp.sum((bins == v).astype(jnp.float32), axis=1, keepdims=True)
    w_b = jnp.sum(jnp.maximum(bins - v, 0.0), axis=1, keepdims=True)
    c_const = jnp.sum(n_b * w_b)
    denom = float(_N * _N) - jnp.sum(n_b * n_b)

    ones_row = jnp.ones((1, _H), dtype=jnp.float32)
    ones_col = jnp.ones((_H, 1), dtype=jnp.float32)
    trow = [jnp.zeros((1, _H), dtype=jnp.float32) for _ in range(_K)]
    tcol = [jnp.zeros((_H, 1), dtype=jnp.float32) for _ in range(_K)]
    for q in range(_K):
        rr = rrow[:, q * _H:(q + 1) * _H]
        for p in range(q + 1):
            tb = jnp.tanh(rr - rcol[p * _H:(p + 1) * _H, :])
            trow[q] = trow[q] + jax.lax.dot_general(
                ones_row, tb, (((1,), (0,)), ((), ())),
                preferred_element_type=jnp.float32)
            if p < q:
                tcol[p] = tcol[p] - jax.lax.dot_general(
                    tb, ones_col, (((1,), (0,)), ((), ())),
                    preferred_element_type=jnp.float32)

    num = c_const
    for p in range(_K):
        t_p = trow[p] + jnp.transpose(tcol[p], (1, 0))
        num = num + jnp.sum(v[:, p * _H:(p + 1) * _H] * t_p)
    out_ref[0, 0] = num / jnp.maximum(denom, 1.0)


def kernel(z_hyp, batch_indices):
    loss = pl.pallas_call(
        _rank_loss_kernel,
        in_specs=[
            pl.BlockSpec((_N, 128), lambda: (0, 0)),
            pl.BlockSpec((1, _N), lambda: (0, 0)),
        ],
        out_specs=pl.BlockSpec(block_shape=(1, 1), index_map=lambda: (0, 0),
                               memory_space=pltpu.SMEM),
        out_shape=jax.ShapeDtypeStruct((1, 1), jnp.float32),
    )(z_hyp, batch_indices.reshape(1, _N))
    return loss[0, 0]
